# baseline (device time: 20896 ns/iter reference)
import functools

import jax
import jax.numpy as jnp
from jax import lax
from jax.experimental import pallas as pl
from jax.experimental.pallas import tpu as pltpu

N_DEV = 4
TOK = 256
D = 128
H = 256
E_PER = 2
N_EXP = 8
CAPACITY = 102


def kernel(x, router_W, route_idx, expert_W):
    del router_W

    def body(x_ref, r_ref, ew_ref, out_ref,
             comm_w, comm_r, send_w, recv_w, send_r, recv_r):
        my = lax.axis_index("i")
        left = (my - 1) % N_DEV
        right = (my + 1) % N_DEV

        barrier_sem = pltpu.get_barrier_semaphore()
        for nbr in (left, right):
            pl.semaphore_signal(
                barrier_sem, inc=1,
                device_id=(nbr,), device_id_type=pl.DeviceIdType.MESH,
            )
        pl.semaphore_wait(barrier_sem, 2)

        comm_w[0] = ew_ref[...].astype(jnp.bfloat16)
        comm_r[0] = r_ref[...]

        xb = x_ref[...].astype(jnp.bfloat16)
        r = r_ref[...]

        acc = jnp.zeros((TOK, H), jnp.float32)
        for k in range(E_PER):
            e = E_PER * my + k
            g = (r == e).astype(jnp.bfloat16)
            acc = acc + jnp.dot(xb * g, comm_w[0, k],
                                preferred_element_type=jnp.float32)

        pre = [jnp.float32(0.0)] * N_EXP

        for h in range(N_DEV - 1):
            s_slot = h % 2
            r_slot = (h + 1) % 2
            rd_w = pltpu.make_async_remote_copy(
                src_ref=comm_w.at[s_slot], dst_ref=comm_w.at[r_slot],
                send_sem=send_w.at[s_slot], recv_sem=recv_w.at[r_slot],
                device_id=(right,), device_id_type=pl.DeviceIdType.MESH,
            )
            rd_r = pltpu.make_async_remote_copy(
                src_ref=comm_r.at[s_slot], dst_ref=comm_r.at[r_slot],
                send_sem=send_r.at[s_slot], recv_sem=recv_r.at[r_slot],
                device_id=(right,), device_id_type=pl.DeviceIdType.MESH,
            )
            rd_w.start()
            rd_r.start()
            rd_w.wait()
            rd_r.wait()

            origin = (my - h - 1) % N_DEV
            chunk_r = comm_r[r_slot]
            before = (origin < my).astype(jnp.float32)
            for e in range(N_EXP):
                pre[e] = pre[e] + before * jnp.sum(
                    (chunk_r == e).astype(jnp.float32))
            for k in range(E_PER):
                e = E_PER * origin + k
                g = (r == e).astype(jnp.bfloat16)
                acc = acc + jnp.dot(xb * g, comm_w[r_slot, k],
                                    preferred_element_type=jnp.float32)

        eiota = lax.broadcasted_iota(jnp.int32, (TOK, N_EXP), 1)
        M = (r == eiota).astype(jnp.float32)
        ii = lax.broadcasted_iota(jnp.int32, (TOK, TOK), 0)
        jj = lax.broadcasted_iota(jnp.int32, (TOK, TOK), 1)
        L = (ii > jj).astype(jnp.float32)
        C = jnp.dot(L, M, preferred_element_type=jnp.float32)
        rank_local = jnp.sum(M * C, axis=1, keepdims=True)
        pre_tok = jnp.zeros((TOK, 1), jnp.float32)
        for e in range(N_EXP):
            pre_tok = pre_tok + M[:, e:e + 1] * pre[e]
        kept = ((rank_local + pre_tok) < float(CAPACITY)).astype(jnp.float32)

        out_ref[...] = acc * kept

    return pl.pallas_call(
        body,
        out_shape=jax.ShapeDtypeStruct((TOK, H), jnp.float32),
        in_specs=[
            pl.BlockSpec(memory_space=pltpu.VMEM),
            pl.BlockSpec(memory_space=pltpu.VMEM),
            pl.BlockSpec(memory_space=pltpu.VMEM),
        ],
        out_specs=pl.BlockSpec(memory_space=pltpu.VMEM),
        scratch_shapes=[
            pltpu.VMEM((2, E_PER, D, H), jnp.bfloat16),
            pltpu.VMEM((2, TOK, 1), jnp.int32),
            pltpu.SemaphoreType.DMA((2,)),
            pltpu.SemaphoreType.DMA((2,)),
            pltpu.SemaphoreType.DMA((2,)),
            pltpu.SemaphoreType.DMA((2,)),
        ],
        compiler_params=pltpu.CompilerParams(collective_id=0),
    )(x, route_idx, expert_W)


# device time: 11200 ns/iter; 1.8657x vs baseline; 1.8657x over previous
import jax
import jax.numpy as jnp
from jax import lax
from jax.experimental import pallas as pl
from jax.experimental.pallas import tpu as pltpu

N_DEV = 4
TOK = 256
D = 128
H = 256
E_PER = 2
N_EXP = 8
CAPACITY = 102
ROWS = E_PER * D + 1


def kernel(x, router_W, route_idx, expert_W):
    del router_W

    def body(x_ref, r_ref, ew_ref, out_ref, stage, rbuf, send_sems, recv_sems):
        my = lax.axis_index("i")

        barrier_sem = pltpu.get_barrier_semaphore()
        for d in range(1, N_DEV):
            pl.semaphore_signal(
                barrier_sem, inc=1,
                device_id=((my + d) % N_DEV,),
                device_id_type=pl.DeviceIdType.MESH,
            )

        xb = x_ref[...].astype(jnp.bfloat16)
        r = r_ref[...]

        lane = lax.broadcasted_iota(jnp.int32, (TOK, H), 1)
        M = (r == lane).astype(jnp.float32)
        hist_row = jnp.sum(M, axis=0, keepdims=True)

        stage[0:D, :] = ew_ref[0].astype(jnp.bfloat16)
        stage[D:2 * D, :] = ew_ref[1].astype(jnp.bfloat16)
        stage[2 * D:2 * D + 1, :] = hist_row.astype(jnp.bfloat16)

        pl.semaphore_wait(barrier_sem, N_DEV - 1)

        sends = []
        for d in range(1, N_DEV):
            rd = pltpu.make_async_remote_copy(
                src_ref=stage, dst_ref=rbuf.at[N_DEV - d - 1],
                send_sem=send_sems.at[d - 1], recv_sem=recv_sems.at[N_DEV - d - 1],
                device_id=((my + d) % N_DEV,),
                device_id_type=pl.DeviceIdType.MESH,
            )
            rd.start()
            sends.append(rd)

        acc = jnp.zeros((TOK, H), jnp.float32)
        for k in range(E_PER):
            e = E_PER * my + k
            g = (r == e).astype(jnp.bfloat16)
            acc = acc + jnp.dot(xb * g, stage[k * D:(k + 1) * D, :],
                                preferred_element_type=jnp.float32)

        ii = lax.broadcasted_iota(jnp.int32, (TOK, TOK), 0)
        jj = lax.broadcasted_iota(jnp.int32, (TOK, TOK), 1)
        Lt = (ii > jj).astype(jnp.float32)
        C = jnp.dot(Lt, M, preferred_element_type=jnp.float32)
        rank_local = jnp.sum(M * C, axis=1, keepdims=True)

        pre_vec = jnp.zeros((1, H), jnp.float32)
        for j in (0, 2, 1):
            recv = pltpu.make_async_remote_copy(
                src_ref=stage, dst_ref=rbuf.at[j],
                send_sem=send_sems.at[0], recv_sem=recv_sems.at[j],
                device_id=(my,), device_id_type=pl.DeviceIdType.MESH,
            )
            recv.wait_recv()
            o = (my + j + 1) % N_DEV
            before = (o < my).astype(jnp.float32)
            hist_o = rbuf[j, 2 * D:2 * D + 1, :].astype(jnp.float32)
            pre_vec = pre_vec + before * hist_o
            for k in range(E_PER):
                e = E_PER * o + k
                g = (r == e).astype(jnp.bfloat16)
                acc = acc + jnp.dot(xb * g, rbuf[j, k * D:(k + 1) * D, :],
                                    preferred_element_type=jnp.float32)

        pre_tok = jnp.sum(M * pre_vec, axis=1, keepdims=True)
        kept = ((rank_local + pre_tok) < float(CAPACITY)).astype(jnp.float32)
        out_ref[...] = acc * kept

        for rd in sends:
            rd.wait_send()

    return pl.pallas_call(
        body,
        out_shape=jax.ShapeDtypeStruct((TOK, H), jnp.float32),
        in_specs=[
            pl.BlockSpec(memory_space=pltpu.VMEM),
            pl.BlockSpec(memory_space=pltpu.VMEM),
            pl.BlockSpec(memory_space=pltpu.VMEM),
        ],
        out_specs=pl.BlockSpec(memory_space=pltpu.VMEM),
        scratch_shapes=[
            pltpu.VMEM((ROWS, H), jnp.bfloat16),
            pltpu.VMEM((N_DEV - 1, ROWS, H), jnp.bfloat16),
            pltpu.SemaphoreType.DMA((N_DEV - 1,)),
            pltpu.SemaphoreType.DMA((N_DEV - 1,)),
        ],
        compiler_params=pltpu.CompilerParams(collective_id=0),
    )(x, route_idx, expert_W)
